# Initial kernel scaffold; baseline (speedup 1.0000x reference)
import jax
import jax.numpy as jnp
from jax.experimental import pallas as pl


def kernel(h, w, b):
    raise NotImplementedError("write your pallas kernel here")


# speedup vs baseline: 1.0161x; 1.0161x over previous
import functools

import jax
import jax.numpy as jnp
from jax.experimental import pallas as pl
from jax.experimental.pallas import tpu as pltpu


def _round_up(x, m):
    return ((x + m - 1) // m) * m


def _pool_kernel(h_ref, w_ref, b_ref, out_ref, *, inv_n):
    tg, n, f = h_ref.shape
    h2 = h_ref[...].reshape(tg * n, f)
    s = jax.lax.dot(h2, w_ref[...], preferred_element_type=jnp.float32)
    s = s + b_ref[0, 0]
    a = jnp.where(s > 0, s, 0.01 * s)
    att = jnp.exp(a)
    wt = att * h2.astype(jnp.float32)
    acc = jnp.sum(wt.reshape(tg, n, f), axis=1)
    out_ref[...] = (acc * inv_n).astype(out_ref.dtype)


def _readout(h, w, b, *, block_graphs):
    G, N, F = h.shape

    Np = _round_up(N, 8)
    if Np != N:
        h = jnp.pad(h, ((0, 0), (0, Np - N), (0, 0)))
    TG = min(block_graphs, _round_up(G, 8))
    Gp = _round_up(G, TG)
    if Gp != G:
        h = jnp.pad(h, ((0, Gp - G), (0, 0), (0, 0)))

    w_rep = jnp.broadcast_to(w.reshape(F, 1), (F, F)).astype(h.dtype)
    b2 = b.reshape(1, 1).astype(jnp.float32)

    body = functools.partial(_pool_kernel, inv_n=1.0 / float(N))

    out = pl.pallas_call(
        body,
        out_shape=jax.ShapeDtypeStruct((Gp, F), jnp.float32),
        grid_spec=pltpu.PrefetchScalarGridSpec(
            num_scalar_prefetch=0,
            grid=(Gp // TG,),
            in_specs=[
                pl.BlockSpec((TG, Np, F), lambda g: (g, 0, 0)),
                pl.BlockSpec((F, F), lambda g: (0, 0)),
                pl.BlockSpec(memory_space=pltpu.MemorySpace.SMEM),
            ],
            out_specs=pl.BlockSpec((TG, F), lambda g: (g, 0)),
        ),
        compiler_params=pltpu.CompilerParams(
            dimension_semantics=("parallel",),
            vmem_limit_bytes=64 * 1024 * 1024,
        ),
    )(h, w_rep, b2)

    return out[:G]


def kernel(h, w, b):
    return _readout(h, w, b, block_graphs=64)


# speedup vs baseline: 1.2871x; 1.2667x over previous
import functools

import jax
import jax.numpy as jnp
from jax.experimental import pallas as pl
from jax.experimental.pallas import tpu as pltpu


def _round_up(x, m):
    return ((x + m - 1) // m) * m


def _pool_kernel(h_ref, w_ref, b_ref, out_ref, *, inv_n):
    tg, n, f = h_ref.shape
    h2 = h_ref[...].reshape(tg * n, f)
    s = jax.lax.dot(h2, w_ref[...], preferred_element_type=jnp.float32)
    s = s + b_ref[0, 0]
    a = jnp.where(s > 0, s, 0.01 * s)
    att = jnp.exp(a)
    wt = att * h2.astype(jnp.float32)
    acc = jnp.sum(wt.reshape(tg, n, f), axis=1)
    out_ref[...] = (acc * inv_n).astype(out_ref.dtype)


def _readout(h, w, b, *, block_graphs):
    G, N, F = h.shape

    Np = _round_up(N, 8)
    if Np != N:
        h = jnp.pad(h, ((0, 0), (0, Np - N), (0, 0)))
    TG = min(block_graphs, _round_up(G, 8))
    Gp = _round_up(G, TG)
    if Gp != G:
        h = jnp.pad(h, ((0, Gp - G), (0, 0), (0, 0)))
    steps = Gp // TG

    w_rep = jnp.broadcast_to(w.reshape(F, 1), (F, F)).astype(h.dtype)
    b2 = b.reshape(1, 1).astype(jnp.float32)

    body = functools.partial(_pool_kernel, inv_n=1.0 / float(N))

    out = pl.pallas_call(
        body,
        out_shape=jax.ShapeDtypeStruct((Gp, F), jnp.float32),
        grid_spec=pltpu.PrefetchScalarGridSpec(
            num_scalar_prefetch=0,
            grid=(steps,),
            in_specs=[
                pl.BlockSpec((TG, Np, F), lambda s: (s, 0, 0)),
                pl.BlockSpec((F, F), lambda s: (0, 0)),
                pl.BlockSpec(memory_space=pltpu.MemorySpace.SMEM),
            ],
            out_specs=pl.BlockSpec((TG, F), lambda s: (s, 0)),
        ),
        compiler_params=pltpu.CompilerParams(
            dimension_semantics=("parallel",),
            vmem_limit_bytes=64 * 1024 * 1024,
        ),
    )(h, w_rep, b2)

    return out[:G]


def kernel(h, w, b):
    return _readout(h, w, b, block_graphs=128)


# speedup vs baseline: 1.4330x; 1.1133x over previous
import functools

import jax
import jax.numpy as jnp
from jax.experimental import pallas as pl
from jax.experimental.pallas import tpu as pltpu


def _round_up(x, m):
    return ((x + m - 1) // m) * m


def _pool_kernel(h_ref, w_ref, b_ref, out_ref, *, inv_n):
    tg, n, f = h_ref.shape
    h2 = h_ref[...].reshape(tg * n, f)
    s = jax.lax.dot(h2, w_ref[...], preferred_element_type=jnp.float32)
    s = s + b_ref[0, 0]
    a = jnp.where(s > 0, s, 0.01 * s)
    att = jnp.exp(a)
    wt = att * h2.astype(jnp.float32)
    acc = jnp.sum(wt.reshape(tg, n, f), axis=1)
    out_ref[...] = (acc * inv_n).astype(out_ref.dtype)


def _readout(h, w, b, *, block_graphs):
    G, N, F = h.shape

    Np = _round_up(N, 8)
    if Np != N:
        h = jnp.pad(h, ((0, 0), (0, Np - N), (0, 0)))
    TG = min(block_graphs, _round_up(G, 8))
    Gp = _round_up(G, TG)
    if Gp != G:
        h = jnp.pad(h, ((0, Gp - G), (0, 0), (0, 0)))
    steps = Gp // TG

    w_rep = jnp.broadcast_to(w.reshape(F, 1), (F, F)).astype(h.dtype)
    b2 = b.reshape(1, 1).astype(jnp.float32)

    body = functools.partial(_pool_kernel, inv_n=1.0 / float(N))

    out = pl.pallas_call(
        body,
        out_shape=jax.ShapeDtypeStruct((Gp, F), jnp.float32),
        grid_spec=pltpu.PrefetchScalarGridSpec(
            num_scalar_prefetch=0,
            grid=(steps,),
            in_specs=[
                pl.BlockSpec((TG, Np, F), lambda s: (s, 0, 0)),
                pl.BlockSpec((F, F), lambda s: (0, 0)),
                pl.BlockSpec(memory_space=pltpu.MemorySpace.SMEM),
            ],
            out_specs=pl.BlockSpec((TG, F), lambda s: (s, 0)),
        ),
        compiler_params=pltpu.CompilerParams(
            dimension_semantics=("parallel",),
            vmem_limit_bytes=64 * 1024 * 1024,
        ),
    )(h, w_rep, b2)

    return out[:G]


def kernel(h, w, b):
    return _readout(h, w, b, block_graphs=256)


# speedup vs baseline: 1.4443x; 1.0079x over previous
import functools

import jax
import jax.numpy as jnp
from jax.experimental import pallas as pl
from jax.experimental.pallas import tpu as pltpu


def _round_up(x, m):
    return ((x + m - 1) // m) * m


def _pool_kernel(h_ref, w_ref, b_ref, out_ref, *, inv_n):
    tg, n, f = h_ref.shape
    h2 = h_ref[...].reshape(tg * n, f)
    s = jax.lax.dot(h2, w_ref[...], preferred_element_type=jnp.float32)
    s = s + b_ref[0, 0]
    a = jnp.where(s > 0, s, 0.01 * s)
    att = jnp.exp(a)
    wt = att * h2.astype(jnp.float32)
    acc = jnp.sum(wt.reshape(tg, n, f), axis=1)
    out_ref[...] = (acc * inv_n).astype(out_ref.dtype)


def _readout(h, w, b, *, block_graphs):
    G, N, F = h.shape

    Np = _round_up(N, 8)
    if Np != N:
        h = jnp.pad(h, ((0, 0), (0, Np - N), (0, 0)))
    TG = min(block_graphs, _round_up(G, 8))
    Gp = _round_up(G, TG)
    if Gp != G:
        h = jnp.pad(h, ((0, Gp - G), (0, 0), (0, 0)))
    steps = Gp // TG

    w_rep = jnp.broadcast_to(w.reshape(F, 1), (F, F)).astype(h.dtype)
    b2 = b.reshape(1, 1).astype(jnp.float32)

    body = functools.partial(_pool_kernel, inv_n=1.0 / float(N))

    out = pl.pallas_call(
        body,
        out_shape=jax.ShapeDtypeStruct((Gp, F), jnp.float32),
        grid_spec=pltpu.PrefetchScalarGridSpec(
            num_scalar_prefetch=0,
            grid=(steps,),
            in_specs=[
                pl.BlockSpec((TG, Np, F), lambda s: (s, 0, 0)),
                pl.BlockSpec((F, F), lambda s: (0, 0)),
                pl.BlockSpec(memory_space=pltpu.MemorySpace.SMEM),
            ],
            out_specs=pl.BlockSpec((TG, F), lambda s: (s, 0)),
        ),
        compiler_params=pltpu.CompilerParams(
            dimension_semantics=("parallel",),
            vmem_limit_bytes=64 * 1024 * 1024,
        ),
    )(h, w_rep, b2)

    return out[:G]


def kernel(h, w, b):
    return _readout(h, w, b, block_graphs=512)
